# baseline (speedup 1.0000x reference)
import dataclasses
import functools

import jax
import jax.numpy as jnp
from jax import lax
from jax.experimental import pallas as pl
from jax.experimental.pallas import tpu as pltpu
from jax.experimental.pallas import tpu_sc as plsc

VOCAB = 1000000
D = 64
B = 4096
S = 200
EPS = 1e-5

CHUNK = 128
NBUF = 4
NOUT = 2
N_WORKERS = 32
TOTAL_ROWS = B * S
N_CHUNKS = TOTAL_ROWS // CHUNK
CHUNKS_PER_W = N_CHUNKS // N_WORKERS
UNROLL = 4


def _rsqrt_nr(x16):
    i = plsc.bitcast(x16, jnp.int32)
    y = plsc.bitcast(jnp.int32(0x5F3759DF) - (i >> 1), jnp.float32)
    y = y * (1.5 - 0.5 * x16 * y * y)
    y = y * (1.5 - 0.5 * x16 * y * y)
    return y


def _ln_rows(rin, rout, gvecs, bvecs):
    @pl.loop(0, CHUNK, step=UNROLL)
    def _(r0):
        for dr in range(UNROLL):
            r = r0 + dr
            vs = [rin[r, pl.ds(16 * q, 16)] for q in range(4)]
            sv = (vs[0] + vs[1]) + (vs[2] + vs[3])
            qv = (vs[0] * vs[0] + vs[1] * vs[1]) + (vs[2] * vs[2] + vs[3] * vs[3])
            tot = jnp.sum(sv)
            qtot = jnp.sum(qv)
            mean = tot * (1.0 / 64.0)
            var = qtot * (1.0 / 64.0) - mean * mean
            xv = jnp.broadcast_to(var + EPS, (16,))
            rstd = _rsqrt_nr(xv)
            for q in range(4):
                outv = (vs[q] - mean) * rstd * gvecs[q] + bvecs[q]
                rout[r, pl.ds(16 * q, 16)] = outv


def _sc_embed_ln(tok2d, table, gamma, beta):
    mesh = plsc.VectorSubcoreMesh(core_axis_name="c", subcore_axis_name="s")
    cp = pltpu.CompilerParams()
    for fld, val in (("needs_layout_passes", False),
                     ("use_tc_tiling_on_sc", False)):
        if fld in pltpu.CompilerParams.__dataclass_fields__:
            cp = dataclasses.replace(cp, **{fld: val})

    @functools.partial(
        pl.kernel,
        mesh=mesh,
        compiler_params=cp,
        out_type=jax.ShapeDtypeStruct((TOTAL_ROWS, 2 * D), jnp.float32),
        scratch_types=(
            [pltpu.VMEM((CHUNKS_PER_W, CHUNK), jnp.int32)]
            + [pltpu.VMEM((CHUNK, 2 * D), jnp.float32)
               for _ in range(NBUF + NOUT)]
            + [pltpu.VMEM((D,), jnp.float32) for _ in range(2)]
            + [pltpu.SemaphoreType.DMA for _ in range(NBUF + NOUT)]
        ),
    )
    def k(tok_hbm, table_hbm, gamma_hbm, beta_hbm, out_hbm, *scratch):
        idx_v = scratch[0]
        rin = list(scratch[1:1 + NBUF])
        rout = list(scratch[1 + NBUF:1 + NBUF + NOUT])
        gamma_v, beta_v = scratch[1 + NBUF + NOUT:3 + NBUF + NOUT]
        gsem = list(scratch[3 + NBUF + NOUT:3 + 2 * NBUF + NOUT])
        osem = list(scratch[3 + 2 * NBUF + NOUT:])

        wid = lax.axis_index("c") * 16 + lax.axis_index("s")
        base_chunk = wid * CHUNKS_PER_W

        pltpu.sync_copy(tok_hbm.at[pl.ds(base_chunk, CHUNKS_PER_W)], idx_v)
        pltpu.sync_copy(gamma_hbm, gamma_v)
        pltpu.sync_copy(beta_hbm, beta_v)
        gvecs = [gamma_v[pl.ds(16 * q, 16)] for q in range(4)]
        bvecs = [beta_v[pl.ds(16 * q, 16)] for q in range(4)]

        for b in range(NBUF):
            pltpu.async_copy(table_hbm.at[idx_v.at[b]], rin[b], gsem[b])

        @pl.loop(0, CHUNKS_PER_W, step=NBUF)
        def _(s0):
            for b in range(NBUF):
                s = s0 + b
                o = b % NOUT

                pltpu.make_async_copy(
                    table_hbm.at[idx_v.at[s]], rin[b], gsem[b]
                ).wait()

                @pl.when(s >= NOUT)
                def _():
                    pltpu.make_async_copy(
                        rout[o], out_hbm.at[pl.ds(0, CHUNK)], osem[o]
                    ).wait()

                _ln_rows(rin[b], rout[o], gvecs, bvecs)

                pltpu.async_copy(
                    rout[o],
                    out_hbm.at[pl.ds((base_chunk + s) * CHUNK, CHUNK)],
                    osem[o],
                )

                @pl.when(s + NBUF < CHUNKS_PER_W)
                def _():
                    pltpu.async_copy(
                        table_hbm.at[idx_v.at[s + NBUF]], rin[b], gsem[b]
                    )

        for o in range(NOUT):
            pltpu.make_async_copy(
                rout[o], out_hbm.at[pl.ds(0, CHUNK)], osem[o]
            ).wait()

    return k(tok2d, table, gamma, beta)


def kernel(token_ids, table, gamma, beta):
    tok2d = jnp.reshape(token_ids, (N_CHUNKS, CHUNK)).astype(jnp.int32)
    tblp = jnp.concatenate([table, table], axis=1)
    out = _sc_embed_ln(tok2d, tblp, gamma, beta)
    return jnp.reshape(out[:, :D], (B, S, D))
